# Initial kernel scaffold; baseline (speedup 1.0000x reference)
import jax
import jax.numpy as jnp
from jax.experimental import pallas as pl


def kernel(x, up, gate, down, router_w):
    raise NotImplementedError("write your pallas kernel here")


# speedup vs baseline: 1.1258x; 1.1258x over previous
import functools

import jax
import jax.numpy as jnp
from jax.experimental import pallas as pl
from jax.experimental.pallas import tpu as pltpu

E = 8
SHARED = 1
TOPK = 2
D = 1024
DFF = 2048
NR = E - SHARED


def _router_body(x_ref, rw_ref, w_ref):
    logits = jax.lax.dot_general(
        x_ref[...], rw_ref[...],
        (((1,), (1,)), ((), ())),
        preferred_element_type=jnp.float32,
    )
    m = jnp.max(logits, axis=-1, keepdims=True)
    ex = jnp.exp(logits - m)
    probs = ex / jnp.sum(ex, axis=-1, keepdims=True)

    col = jax.lax.broadcasted_iota(jnp.int32, probs.shape, 1)
    v1 = jnp.max(probs, axis=-1, keepdims=True)
    i1 = jnp.min(jnp.where(probs == v1, col, NR), axis=-1, keepdims=True)
    pm = jnp.where(col == i1, -jnp.inf, probs)
    v2 = jnp.max(pm, axis=-1, keepdims=True)
    i2 = jnp.min(jnp.where(pm == v2, col, NR), axis=-1, keepdims=True)
    denom = v1 + v2 + 1e-9
    w_routed = (jnp.where(col == i1, v1, 0.0)
                + jnp.where(col == i2, v2, 0.0)) / denom
    shared_w = jnp.full((x_ref.shape[0], SHARED), 1.0 / SHARED, jnp.float32)
    w_ref[...] = jnp.concatenate([shared_w, w_routed], axis=-1)


def _router_weights(flat, router_w):
    T = flat.shape[0]
    return pl.pallas_call(
        _router_body,
        out_shape=jax.ShapeDtypeStruct((T, E), jnp.float32),
    )(flat, router_w)


def _ffn_body(w_ref, x_ref, up_ref, gate_ref, down_ref, out_ref, acc_ref):
    e = pl.program_id(0)
    t = pl.program_id(1)
    bt = x_ref.shape[0]

    xb = x_ref[...]
    u = jax.lax.dot_general(
        xb, up_ref[0], (((1,), (1,)), ((), ())),
        preferred_element_type=jnp.float32)
    g = jax.lax.dot_general(
        xb, gate_ref[0], (((1,), (1,)), ((), ())),
        preferred_element_type=jnp.float32)
    h = (g * jax.nn.sigmoid(g) * u).astype(jnp.bfloat16)
    o = jax.lax.dot_general(
        h, down_ref[0], (((1,), (1,)), ((), ())),
        preferred_element_type=jnp.float32)

    ecol = jax.lax.broadcasted_iota(jnp.int32, w_ref.shape, 1)
    wcol = jnp.sum(jnp.where(ecol == e, w_ref[...], 0.0), axis=1,
                   keepdims=True)
    contrib = wcol * o

    sl = pl.ds(t * bt, bt)

    @pl.when(e == 0)
    def _():
        acc_ref[sl, :] = contrib

    @pl.when(e > 0)
    def _():
        acc_ref[sl, :] = acc_ref[sl, :] + contrib

    @pl.when(e == E - 1)
    def _():
        out_ref[...] = acc_ref[sl, :]


def _moe_ffn(flat_bf16, w, up, gate, down):
    T = flat_bf16.shape[0]
    BT = 512
    grid = (E, T // BT)
    return pl.pallas_call(
        _ffn_body,
        grid=grid,
        in_specs=[
            pl.BlockSpec((BT, E), lambda e, t: (t, 0)),
            pl.BlockSpec((BT, D), lambda e, t: (t, 0)),
            pl.BlockSpec((1, DFF, D), lambda e, t: (e, 0, 0)),
            pl.BlockSpec((1, DFF, D), lambda e, t: (e, 0, 0)),
            pl.BlockSpec((1, D, DFF), lambda e, t: (e, 0, 0)),
        ],
        out_specs=pl.BlockSpec((BT, D), lambda e, t: (t, 0)),
        out_shape=jax.ShapeDtypeStruct((T, D), jnp.float32),
        scratch_shapes=[pltpu.VMEM((T, D), jnp.float32)],
        compiler_params=pltpu.CompilerParams(
            dimension_semantics=("arbitrary", "arbitrary"),
        ),
    )(w, flat_bf16, up, gate, down)


def kernel(x, up, gate, down, router_w):
    orig_shape = x.shape
    flat = x.reshape(-1, D)
    w = _router_weights(flat, router_w)
    out = _moe_ffn(flat.astype(jnp.bfloat16), w,
                   up.astype(jnp.bfloat16),
                   gate.astype(jnp.bfloat16),
                   down.astype(jnp.bfloat16))
    return out.reshape(orig_shape)
